# Initial kernel scaffold; baseline (speedup 1.0000x reference)
import jax
import jax.numpy as jnp
from jax.experimental import pallas as pl


def kernel(x):
    raise NotImplementedError("write your pallas kernel here")


# speedup vs baseline: 10.2539x; 10.2539x over previous
import functools

import jax
import jax.numpy as jnp
from jax import lax
from jax.experimental import pallas as pl
from jax.experimental.pallas import tpu as pltpu
from jax.experimental.pallas import tpu_sc as plsc

_B = 4
_S = 4
_D = 2048
_NC = 2
_NS = 16
_NW = _NC * _NS
_CHUNK = (_B * _S * _D) // _NW
_PER_ROW = _D // _CHUNK

_mesh = plsc.VectorSubcoreMesh(core_axis_name="c", subcore_axis_name="s")


@functools.partial(
    pl.kernel,
    mesh=_mesh,
    out_type=jax.ShapeDtypeStruct((_B, _S, _D), jnp.float32),
    scratch_types=[pltpu.VMEM((_CHUNK,), jnp.float32)],
)
def _gather_head(x_hbm, out_hbm, buf):
    wid = lax.axis_index("s") * _NC + lax.axis_index("c")
    row = wid // _PER_ROW
    part = wid % _PER_ROW
    b = row // _S
    s = row % _S
    off = part * _CHUNK
    pltpu.sync_copy(x_hbm.at[b, s, 0, pl.ds(off, _CHUNK)], buf)
    pltpu.sync_copy(buf, out_hbm.at[b, s, pl.ds(off, _CHUNK)])


def kernel(x):
    return _gather_head(x)


# speedup vs baseline: 11.0394x; 1.0766x over previous
import functools

import jax
import jax.numpy as jnp
from jax import lax
from jax.experimental import pallas as pl
from jax.experimental.pallas import tpu as pltpu
from jax.experimental.pallas import tpu_sc as plsc

_B = 4
_S = 4
_D = 2048
_NC = 1
_NS = 16
_NW = _NC * _NS
_CHUNK = (_B * _S * _D) // _NW
_PER_ROW = _D // _CHUNK

_mesh = plsc.VectorSubcoreMesh(
    core_axis_name="c", subcore_axis_name="s", num_cores=1
)


@functools.partial(
    pl.kernel,
    mesh=_mesh,
    out_type=jax.ShapeDtypeStruct((_B, _S, _D), jnp.float32),
    scratch_types=[pltpu.VMEM((_CHUNK,), jnp.float32)],
)
def _gather_head(x_hbm, out_hbm, buf):
    wid = lax.axis_index("s") * _NC + lax.axis_index("c")
    row = wid // _PER_ROW
    part = wid % _PER_ROW
    b = row // _S
    s = row % _S
    off = part * _CHUNK
    pltpu.sync_copy(x_hbm.at[b, s, 0, pl.ds(off, _CHUNK)], buf)
    pltpu.sync_copy(buf, out_hbm.at[b, s, pl.ds(off, _CHUNK)])


def kernel(x):
    return _gather_head(x)


# speedup vs baseline: 11.8459x; 1.0731x over previous
import functools

import jax
import jax.numpy as jnp
from jax import lax
from jax.experimental import pallas as pl
from jax.experimental.pallas import tpu as pltpu
from jax.experimental.pallas import tpu_sc as plsc

_B = 4
_S = 4
_D = 2048
_NC = 1
_NS = 16
_NW = _NC * _NS
_CHUNK = (_B * _S * _D) // _NW
_PER_ROW = _D // _CHUNK

_mesh = plsc.ScalarSubcoreMesh(axis_name="c", num_cores=1)


@functools.partial(
    pl.kernel,
    mesh=_mesh,
    out_type=jax.ShapeDtypeStruct((_B, _S, _D), jnp.float32),
    scratch_types=[
        pltpu.VMEM_SHARED((_B, _S, _D), jnp.float32),
        pltpu.SemaphoreType.DMA,
    ],
)
def _gather_head(x_hbm, out_hbm, stage, sem):
    copies = [
        pltpu.make_async_copy(x_hbm.at[b, s, 0], stage.at[b, s], sem)
        for b in range(_B)
        for s in range(_S)
    ]
    for c in copies:
        c.start()
    for c in copies:
        c.wait()
    pltpu.sync_copy(stage, out_hbm)


def kernel(x):
    return _gather_head(x)
